# baseline (speedup 1.0000x reference)
import functools

import jax
import jax.numpy as jnp
from jax import lax
from jax.experimental import pallas as pl
from jax.experimental.pallas import tpu as pltpu
from jax.experimental.pallas import tpu_sc as plsc

_NC = 2
_NS = 16
_NW = _NC * _NS


def _emb_gather(V, D, N, F, chn):
    n_per_w = N // _NW
    chq = chn * F
    npair = n_per_w // (2 * chn)
    mesh = plsc.VectorSubcoreMesh(core_axis_name="c", subcore_axis_name="s")

    @functools.partial(
        pl.kernel,
        mesh=mesh,
        out_type=jax.ShapeDtypeStruct((N, F, D), jnp.float32),
        compiler_params=pltpu.CompilerParams(use_tc_tiling_on_sc=True),
        scratch_types=[
            pltpu.VMEM((chq,), jnp.int32),
            pltpu.VMEM((chq,), jnp.int32),
            pltpu.VMEM((chq,), jnp.int32),
            pltpu.VMEM((chq,), jnp.int32),
            pltpu.VMEM((chq, 128), jnp.float32),
            pltpu.VMEM((chq, 128), jnp.float32),
            pltpu.VMEM((chn, F, D), jnp.float32),
            pltpu.VMEM((chn, F, D), jnp.float32),
            pltpu.SemaphoreType.DMA,
            pltpu.SemaphoreType.DMA,
            pltpu.SemaphoreType.DMA,
            pltpu.SemaphoreType.DMA,
        ],
    )
    def body(table128, idx4, r4, out,
             ia, ib, ra, rb, rowsa, rowsb, oa, ob, ga, gb, wa, wb):
        wid = lax.axis_index("s") * _NC + lax.axis_index("c")
        qbase = wid * n_per_w * F
        nbase = wid * n_per_w

        def stage_and_fire(q0, idx_v, r_v, rows_v, gsem):
            pltpu.sync_copy(idx4.at[pl.ds(q0, chq)], idx_v)
            pltpu.sync_copy(r4.at[pl.ds(q0, chq)], r_v)
            return pltpu.async_copy(table128.at[idx_v], rows_v, gsem)

        def extract(r_v, rows_v, o_v):
            for g16 in range(chq // 16):
                rv = r_v[pl.ds(g16 * 16, 16)]
                for j in range(16):
                    q = g16 * 16 + j
                    a, f = divmod(q, F)
                    col = rv[j] * D
                    o_v[a, f, pl.ds(0, 16)] = rows_v[q, pl.ds(col, 16)]
                    o_v[a, f, pl.ds(16, 16)] = rows_v[q, pl.ds(col + 16, 16)]

        def pair(c2, _):
            q0 = qbase + c2 * (2 * chq)
            nb0 = nbase + c2 * (2 * chn)
            g0 = stage_and_fire(q0, ia, ra, rowsa, ga)
            g1 = stage_and_fire(q0 + chq, ib, rb, rowsb, gb)
            g0.wait()
            extract(ra, rowsa, oa)
            w0 = pltpu.async_copy(oa, out.at[pl.ds(nb0, chn)], wa)
            g1.wait()
            extract(rb, rowsb, ob)
            w1 = pltpu.async_copy(ob, out.at[pl.ds(nb0 + chn, chn)], wb)
            w0.wait()
            w1.wait()
            return 0

        lax.fori_loop(0, npair, pair, 0)

    return body


def _table_transpose(V, D):
    rpl = 128 // D
    nblk = V // 128
    rows_per_blk = 128 // rpl
    tail = V - nblk * 128
    tail_rows = tail // rpl
    mesh = plsc.VectorSubcoreMesh(core_axis_name="c", subcore_axis_name="s")

    @functools.partial(
        pl.kernel,
        mesh=mesh,
        out_type=jax.ShapeDtypeStruct((V // rpl, 128), jnp.float32),
        compiler_params=pltpu.CompilerParams(
            use_tc_tiling_on_sc=True, needs_layout_passes=False),
        scratch_types=[
            pltpu.VMEM((D, 128), jnp.float32),
            pltpu.VMEM((D, 128), jnp.float32),
            pltpu.VMEM((rows_per_blk, 128), jnp.float32),
            pltpu.VMEM((D, tail), jnp.float32),
            pltpu.VMEM((tail_rows, 128), jnp.float32),
            pltpu.SemaphoreType.DMA,
            pltpu.SemaphoreType.DMA,
        ],
    )
    def body(wt, out, bufa, bufb, oba, buft, obt, wa, wb):
        wid = lax.axis_index("s") * _NC + lax.axis_index("c")
        iota = lax.iota(jnp.int32, 16)
        nb_w = nblk // _NW + jnp.where(wid < (nblk % _NW), 1, 0)

        def transpose_block(buf, o_v, n_rows):
            for s in range(n_rows):
                for j in range(rpl):
                    col = jnp.full((16,), rpl * s + j, jnp.int32)
                    v0 = plsc.load_gather(buf, [iota, col])
                    v1 = plsc.load_gather(buf, [iota + 16, col])
                    o_v[s, pl.ds(j * D, 16)] = v0
                    o_v[s, pl.ds(j * D + 16, 16)] = v1

        def step(c, _):
            blk = wid + c * _NW
            pltpu.sync_copy(wt.at[:, pl.ds(blk * 128, 128)], bufa)
            transpose_block(bufa, oba, rows_per_blk)
            pltpu.async_copy(
                oba, out.at[pl.ds(blk * rows_per_blk, rows_per_blk)],
                wa).wait()
            return 0

        lax.fori_loop(0, nb_w, step, 0)

        if tail:
            @pl.when(wid == _NW - 1)
            def _():
                pltpu.sync_copy(wt.at[:, pl.ds(nblk * 128, tail)], buft)
                transpose_block(buft, obt, tail_rows)
                pltpu.async_copy(
                    obt, out.at[pl.ds(nblk * rows_per_blk, tail_rows)],
                    wb).wait()

    return body


def kernel(weights, indices):
    N, F = indices.shape
    V, D = weights.shape
    rpl = 128 // D
    flat = indices.astype(jnp.int32).reshape(N * F)
    table128 = _table_transpose(V, D)(weights.T)
    return _emb_gather(V, D, N, F, 8)(table128, flat // rpl, flat % rpl)


# speedup vs baseline: 1.7556x; 1.7556x over previous
import functools

import jax
import jax.numpy as jnp
from jax import lax
from jax.experimental import pallas as pl
from jax.experimental.pallas import tpu as pltpu
from jax.experimental.pallas import tpu_sc as plsc

_NC = 2
_NS = 16
_NW = _NC * _NS


def _emb_gather(V, D, N, F, chn):
    n_per_w = N // _NW
    nch = n_per_w // chn
    mesh = plsc.VectorSubcoreMesh(core_axis_name="c", subcore_axis_name="s")

    @functools.partial(
        pl.kernel,
        mesh=mesh,
        out_type=jax.ShapeDtypeStruct((N, F, D), jnp.float32),
        compiler_params=pltpu.CompilerParams(use_tc_tiling_on_sc=False),
        scratch_types=[
            pltpu.VMEM((chn, F), jnp.int32),
            pltpu.VMEM((chn, F), jnp.int32),
            pltpu.VMEM((chn, F, D), jnp.float32),
            pltpu.VMEM((chn, F, D), jnp.float32),
            pltpu.SemaphoreType.DMA,
            pltpu.SemaphoreType.DMA,
            pltpu.SemaphoreType.DMA,
            pltpu.SemaphoreType.DMA,
        ],
    )
    def body(table, idx2, out, i0, i1, r0, r1, g0, g1, w0, w1):
        wid = lax.axis_index("s") * _NC + lax.axis_index("c")
        base = wid * n_per_w
        idx_bufs = (i0, i1)
        row_bufs = (r0, r1)
        gsems = (g0, g1)
        wsems = (w0, w1)

        def fire(s):
            def one(a, _):
                pltpu.async_copy(
                    table.at[idx_bufs[s].at[a]], row_bufs[s].at[a], gsems[s])
                return 0
            lax.fori_loop(0, chn, one, 0)

        def drain(s):
            pltpu.make_async_copy(
                out.at[pl.ds(0, chn)], row_bufs[s], gsems[s]).wait()

        writes = [None] * nch
        pltpu.sync_copy(idx2.at[pl.ds(base, chn)], idx_bufs[0])
        fire(0)
        for c in range(1, nch + 1):
            s = c & 1
            if c < nch:
                pltpu.sync_copy(
                    idx2.at[pl.ds(base + c * chn, chn)], idx_bufs[s])
                if c >= 2:
                    writes[c - 2].wait()
                fire(s)
            p = (c - 1) & 1
            drain(p)
            writes[c - 1] = pltpu.async_copy(
                row_bufs[p], out.at[pl.ds(base + (c - 1) * chn, chn)],
                wsems[p])
        writes[nch - 2].wait()
        writes[nch - 1].wait()

    return body


def kernel(weights, indices):
    N, F = indices.shape
    V, D = weights.shape
    idx = indices.astype(jnp.int32)
    return _emb_gather(V, D, N, F, 64)(weights, idx)
